# baseline (speedup 1.0000x reference)
import functools

import jax
import jax.numpy as jnp
from jax import lax
from jax.experimental import pallas as pl

SCALES = (2, 4, 8, 16)
UNIT = 0.015
LIMIT = 64.0
OFFSET = -0.5 * UNIT * LIMIT

BN = 1024


def _scale_body(pb, pxyz, pp, v_ref, f_ref, scale, bn):
    m = v_ref.shape[1]
    vs = UNIT * scale
    half = 0.5 * vs
    vb = v_ref[0:1, :]
    vx = (v_ref[1:2, :] * vs + OFFSET) + half
    vy = (v_ref[2:3, :] * vs + OFFSET) + half
    vz = (v_ref[3:4, :] * vs + OFFSET) + half
    qq = vx * vx + vy * vy + vz * vz
    vmat = jnp.concatenate([vx, vy, vz], axis=0).astype(jnp.bfloat16)

    dot = jnp.dot(pxyz.astype(jnp.bfloat16), vmat,
                  preferred_element_type=jnp.float32)
    d = (pp + qq) - 2.0 * dot
    d = jnp.maximum(d, 0.0)
    d = jnp.where(pb == vb, d, jnp.float32(1e10))

    iota = lax.broadcasted_iota(jnp.int32, (bn, m), 1)
    wu = jnp.zeros((bn, m), jnp.float32)
    norm = jnp.zeros((bn, 1), jnp.float32)
    for k in range(3):
        mk = jnp.min(d, axis=1, keepdims=True)
        amin = jnp.min(jnp.where(d == mk, iota, m), axis=1, keepdims=True)
        onehot = iota == amin
        rk = 1.0 / (mk + 1e-8)
        wu = wu + jnp.where(onehot, jnp.broadcast_to(rk, (bn, m)), 0.0)
        norm = norm + rk
        if k < 2:
            d = jnp.where(onehot, jnp.float32(1e30), d)
    w = wu * (1.0 / norm)
    return jnp.dot(w, f_ref[...], preferred_element_type=jnp.float32,
                   precision=lax.Precision.HIGHEST)


def _fused_kernel(pts_ref, v1, f1, v2, f2, v3, f3, v4, f4,
                  o1, o2, o3, o4, *, bn):
    pb = pts_ref[:, 0:1]
    px = pts_ref[:, 1:2]
    py = pts_ref[:, 2:3]
    pz = pts_ref[:, 3:4]
    pxyz = pts_ref[:, 1:4]
    pp = px * px + py * py + pz * pz
    for v_ref, f_ref, o_ref, scale in ((v1, f1, o1, SCALES[0]),
                                       (v2, f2, o2, SCALES[1]),
                                       (v3, f3, o3, SCALES[2]),
                                       (v4, f4, o4, SCALES[3])):
        o_ref[...] = _scale_body(pb, pxyz, pp, v_ref, f_ref, scale, bn)


@jax.jit
def kernel(points, batch_ids, feats1_features, feats1_indices,
           feats2_features, feats2_indices, feats3_features, feats3_indices,
           feats4_features, feats4_indices):
    n = points.shape[0]
    pts4 = jnp.concatenate(
        [batch_ids.reshape(-1, 1).astype(jnp.float32), points], axis=1)
    voxes = [jnp.transpose(ii).astype(jnp.float32)
             for ii in (feats1_indices, feats2_indices, feats3_indices,
                        feats4_indices)]
    feats = [feats1_features, feats2_features, feats3_features,
             feats4_features]
    cs = [f.shape[1] for f in feats]
    ms = [v.shape[1] for v in voxes]

    grid = (n // BN,)
    in_specs = [pl.BlockSpec((BN, 4), lambda i: (i, 0))]
    for v, f in zip(voxes, feats):
        in_specs.append(pl.BlockSpec(v.shape, lambda i: (0, 0)))
        in_specs.append(pl.BlockSpec(f.shape, lambda i: (0, 0)))
    out_specs = [pl.BlockSpec((BN, C), lambda i: (i, 0)) for C in cs]
    out_shape = [jax.ShapeDtypeStruct((n, C), jnp.float32) for C in cs]

    args = [pts4]
    for v, f in zip(voxes, feats):
        args.extend((v, f))
    outs = pl.pallas_call(
        functools.partial(_fused_kernel, bn=BN),
        grid=grid,
        in_specs=in_specs,
        out_specs=out_specs,
        out_shape=out_shape,
    )(*args)
    return jnp.concatenate(outs, axis=1)


# speedup vs baseline: 1.7661x; 1.7661x over previous
import functools

import jax
import jax.numpy as jnp
from jax import lax
from jax.experimental import pallas as pl

SCALES = (2, 4, 8, 16)
UNIT = 0.015
LIMIT = 64.0
OFFSET = -0.5 * UNIT * LIMIT

BN = 512


def _scale_body(pb, pxyz, pp, v_ref, f_ref, scale, bn):
    m = v_ref.shape[1]
    vs = UNIT * scale
    half = 0.5 * vs
    vb = v_ref[0:1, :]
    vx = (v_ref[1:2, :] * vs + OFFSET) + half
    vy = (v_ref[2:3, :] * vs + OFFSET) + half
    vz = (v_ref[3:4, :] * vs + OFFSET) + half
    qq = vx * vx + vy * vy + vz * vz
    vmat = jnp.concatenate([vx, vy, vz], axis=0).astype(jnp.bfloat16)

    dot = jnp.dot(pxyz.astype(jnp.bfloat16), vmat,
                  preferred_element_type=jnp.float32)
    d = (pp + qq) - 2.0 * dot
    d = jnp.maximum(d, 0.0)
    d = jnp.where(pb == vb, d, jnp.float32(1e10))

    iota = lax.broadcasted_iota(jnp.int32, (bn, m), 1)
    wu = jnp.zeros((bn, m), jnp.float32)
    norm = jnp.zeros((bn, 1), jnp.float32)
    for k in range(3):
        mk = jnp.min(d, axis=1, keepdims=True)
        amin = jnp.min(jnp.where(d == mk, iota, m), axis=1, keepdims=True)
        onehot = iota == amin
        rk = 1.0 / (mk + 1e-8)
        wu = wu + jnp.where(onehot, jnp.broadcast_to(rk, (bn, m)), 0.0)
        norm = norm + rk
        if k < 2:
            d = jnp.where(onehot, jnp.float32(1e30), d)
    w = wu * (1.0 / norm)
    return jnp.dot(w, f_ref[...], preferred_element_type=jnp.float32,
                   precision=lax.Precision.DEFAULT)


def _fused_kernel(pts_ref, v1, f1, v2, f2, v3, f3, v4, f4,
                  o1, o2, o3, o4, *, bn):
    pb = pts_ref[:, 0:1]
    px = pts_ref[:, 1:2]
    py = pts_ref[:, 2:3]
    pz = pts_ref[:, 3:4]
    pxyz = pts_ref[:, 1:4]
    pp = px * px + py * py + pz * pz
    for v_ref, f_ref, o_ref, scale in ((v1, f1, o1, SCALES[0]),
                                       (v2, f2, o2, SCALES[1]),
                                       (v3, f3, o3, SCALES[2]),
                                       (v4, f4, o4, SCALES[3])):
        o_ref[...] = _scale_body(pb, pxyz, pp, v_ref, f_ref, scale, bn)


@jax.jit
def kernel(points, batch_ids, feats1_features, feats1_indices,
           feats2_features, feats2_indices, feats3_features, feats3_indices,
           feats4_features, feats4_indices):
    n = points.shape[0]
    pts4 = jnp.concatenate(
        [batch_ids.reshape(-1, 1).astype(jnp.float32), points], axis=1)
    voxes = [jnp.transpose(ii).astype(jnp.float32)
             for ii in (feats1_indices, feats2_indices, feats3_indices,
                        feats4_indices)]
    feats = [feats1_features, feats2_features, feats3_features,
             feats4_features]
    cs = [f.shape[1] for f in feats]
    ms = [v.shape[1] for v in voxes]

    grid = (n // BN,)
    in_specs = [pl.BlockSpec((BN, 4), lambda i: (i, 0))]
    for v, f in zip(voxes, feats):
        in_specs.append(pl.BlockSpec(v.shape, lambda i: (0, 0)))
        in_specs.append(pl.BlockSpec(f.shape, lambda i: (0, 0)))
    out_specs = [pl.BlockSpec((BN, C), lambda i: (i, 0)) for C in cs]
    out_shape = [jax.ShapeDtypeStruct((n, C), jnp.float32) for C in cs]

    args = [pts4]
    for v, f in zip(voxes, feats):
        args.extend((v, f))
    outs = pl.pallas_call(
        functools.partial(_fused_kernel, bn=BN),
        grid=grid,
        in_specs=in_specs,
        out_specs=out_specs,
        out_shape=out_shape,
    )(*args)
    return jnp.concatenate(outs, axis=1)


# speedup vs baseline: 2.1835x; 1.2363x over previous
import functools

import jax
import jax.numpy as jnp
from jax import lax
from jax.experimental import pallas as pl

SCALES = (2, 4, 8, 16)
UNIT = 0.015
LIMIT = 64.0
OFFSET = -0.5 * UNIT * LIMIT

BN = 512


def _scale_body(pb, pxyz, pp, v_ref, f_ref, scale, bn):
    m = v_ref.shape[1]
    vs = UNIT * scale
    half = 0.5 * vs
    vb = v_ref[0:1, :]
    vx = (v_ref[1:2, :] * vs + OFFSET) + half
    vy = (v_ref[2:3, :] * vs + OFFSET) + half
    vz = (v_ref[3:4, :] * vs + OFFSET) + half
    qq = vx * vx + vy * vy + vz * vz
    vmat = jnp.concatenate([vx, vy, vz], axis=0).astype(jnp.bfloat16) * 2

    dot2 = jnp.dot(pxyz.astype(jnp.bfloat16), vmat,
                   preferred_element_type=jnp.float32)
    d = (pp + qq) - dot2
    d = jnp.maximum(d, 0.0)
    d = jnp.where(pb == vb, d, jnp.float32(1e10))

    iota = lax.broadcasted_iota(jnp.int32, (bn, m), 1).astype(jnp.float32)
    wu = jnp.zeros((bn, m), jnp.float32)
    norm = jnp.zeros((bn, 1), jnp.float32)
    for k in range(3):
        mk = jnp.min(d, axis=1, keepdims=True)
        amin = jnp.min(jnp.where(d == mk, iota, jnp.float32(m)),
                       axis=1, keepdims=True)
        onehot = iota == amin
        rk = 1.0 / (mk + 1e-8)
        wu = jnp.where(onehot, jnp.broadcast_to(rk, (bn, m)), wu)
        norm = norm + rk
        if k < 2:
            d = jnp.where(onehot, jnp.float32(1e30), d)
    w = wu * (1.0 / norm)
    return jnp.dot(w, f_ref[...], preferred_element_type=jnp.float32,
                   precision=lax.Precision.DEFAULT)


def _fused_kernel(pts_ref, v1, f1, v2, f2, v3, f3, v4, f4,
                  o1, o2, o3, o4, *, bn):
    pb = pts_ref[:, 0:1]
    px = pts_ref[:, 1:2]
    py = pts_ref[:, 2:3]
    pz = pts_ref[:, 3:4]
    pxyz = pts_ref[:, 1:4]
    pp = px * px + py * py + pz * pz
    for v_ref, f_ref, o_ref, scale in ((v1, f1, o1, SCALES[0]),
                                       (v2, f2, o2, SCALES[1]),
                                       (v3, f3, o3, SCALES[2]),
                                       (v4, f4, o4, SCALES[3])):
        o_ref[...] = _scale_body(pb, pxyz, pp, v_ref, f_ref, scale, bn)


@jax.jit
def kernel(points, batch_ids, feats1_features, feats1_indices,
           feats2_features, feats2_indices, feats3_features, feats3_indices,
           feats4_features, feats4_indices):
    n = points.shape[0]
    pts4 = jnp.concatenate(
        [batch_ids.reshape(-1, 1).astype(jnp.float32), points], axis=1)
    voxes = [jnp.transpose(ii).astype(jnp.float32)
             for ii in (feats1_indices, feats2_indices, feats3_indices,
                        feats4_indices)]
    feats = [feats1_features, feats2_features, feats3_features,
             feats4_features]
    cs = [f.shape[1] for f in feats]
    ms = [v.shape[1] for v in voxes]

    grid = (n // BN,)
    in_specs = [pl.BlockSpec((BN, 4), lambda i: (i, 0))]
    for v, f in zip(voxes, feats):
        in_specs.append(pl.BlockSpec(v.shape, lambda i: (0, 0)))
        in_specs.append(pl.BlockSpec(f.shape, lambda i: (0, 0)))
    out_specs = [pl.BlockSpec((BN, C), lambda i: (i, 0)) for C in cs]
    out_shape = [jax.ShapeDtypeStruct((n, C), jnp.float32) for C in cs]

    args = [pts4]
    for v, f in zip(voxes, feats):
        args.extend((v, f))
    outs = pl.pallas_call(
        functools.partial(_fused_kernel, bn=BN),
        grid=grid,
        in_specs=in_specs,
        out_specs=out_specs,
        out_shape=out_shape,
    )(*args)
    return jnp.concatenate(outs, axis=1)
